# Initial kernel scaffold; baseline (speedup 1.0000x reference)
import jax
import jax.numpy as jnp
from jax.experimental import pallas as pl


def kernel(x, edge_index_c2c, edge_index_c2d, edge_index_c2e, edge_index_codoc, edge_index_comention, edge_index_d2e, edge_index_ent, W_neigh, W_self, b):
    raise NotImplementedError("write your pallas kernel here")


# speedup vs baseline: 4.1286x; 4.1286x over previous
import functools

import jax
import jax.numpy as jnp
from jax import lax
from jax.experimental import pallas as pl
from jax.experimental.pallas import tpu as pltpu
from jax.experimental.pallas import tpu_sc as plsc

N_NODES = 10000
D = 128
E_PER_TYPE = 45714
N_TYPES = 7

NC = 2
NS = 16

BLK = 128
NBLK = 23
EPT = NBLK * BLK
E_PAD = NS * EPT
NROWS = 10240
RPT = NROWS // NS
TRASH = N_NODES


def _sc_segment_sums(x, src_r, dst_r, zer128, zer640, one128):
    mesh = plsc.VectorSubcoreMesh(core_axis_name="c", subcore_axis_name="s")

    @functools.partial(
        pl.kernel,
        out_type=[
            jax.ShapeDtypeStruct((N_TYPES, NROWS, D), jnp.float32),
            jax.ShapeDtypeStruct((N_TYPES * NROWS,), jnp.float32),
        ],
        mesh=mesh,
        scratch_types=[
            pltpu.VMEM_SHARED((NROWS, D), jnp.float32),
            pltpu.VMEM_SHARED((NROWS,), jnp.float32),
            pltpu.VMEM((NBLK, BLK), jnp.int32),
            pltpu.VMEM((NBLK, BLK), jnp.int32),
            pltpu.VMEM((BLK, D), jnp.float32),
            pltpu.VMEM((BLK, D), jnp.float32),
            pltpu.VMEM((RPT,), jnp.float32),
            pltpu.VMEM((BLK,), jnp.float32),
            pltpu.SemaphoreType.DMA,
        ],
    )
    def k(x_hbm, src_hbm, dst_hbm, z128_hbm, z640_hbm, o128_hbm,
          acc_out, deg_out,
          acc_s, deg_s, src_v, dst_v, rows_v, z128_v, z640_v, o128_v, sem):
        cid = lax.axis_index("c")
        sid = lax.axis_index("s")
        base = sid * RPT

        pltpu.sync_copy(z128_hbm, z128_v)
        pltpu.sync_copy(z640_hbm, z640_v)
        pltpu.sync_copy(o128_hbm, o128_v)

        for t in range(N_TYPES):
            @pl.when(cid == (t % NC))
            def _():
                for kk in range(RPT // BLK):
                    pltpu.sync_copy(z128_v, acc_s.at[pl.ds(base + kk * BLK, BLK)])
                pltpu.sync_copy(z640_v, deg_s.at[pl.ds(base, RPT)])
                plsc.subcore_barrier()

                pltpu.sync_copy(src_hbm.at[t, sid], src_v)
                pltpu.sync_copy(dst_hbm.at[t, sid], dst_v)

                for j in range(NBLK):
                    pltpu.async_copy(x_hbm.at[src_v.at[j]], rows_v, sem).wait()
                    pltpu.sync_copy(rows_v, acc_s.at[dst_v.at[j]], add=True)
                    pltpu.sync_copy(o128_v, deg_s.at[dst_v.at[j]], add=True)
                plsc.subcore_barrier()

                for kk in range(RPT // BLK):
                    sl = pl.ds(base + kk * BLK, BLK)
                    pltpu.sync_copy(acc_s.at[sl], acc_out.at[t, sl])
                pltpu.sync_copy(deg_s.at[pl.ds(base, RPT)],
                                deg_out.at[pl.ds(t * NROWS + base, RPT)])
                plsc.subcore_barrier()

    return k(x, src_r, dst_r, zer128, zer640, one128)


def _tc_combine(acc, deg3, x, w_neigh, w_self, b):
    R = 1000
    grid = (N_NODES // R,)

    def body(acc_ref, deg_ref, x_ref, wn_ref, ws_ref, b_ref, out_ref):
        degv = deg_ref[...]
        aggn = acc_ref[...] / jnp.maximum(degv, 1.0)
        ws = jnp.sum(ws_ref[...], axis=0)
        res = jnp.dot(x_ref[...], ws, preferred_element_type=jnp.float32)
        res = res + jnp.sum(b_ref[...], axis=0)[None, :]
        for t in range(N_TYPES):
            res = res + jnp.dot(aggn[t], wn_ref[t],
                                preferred_element_type=jnp.float32)
        out_ref[...] = res * (1.0 / N_TYPES)

    return pl.pallas_call(
        body,
        grid=grid,
        in_specs=[
            pl.BlockSpec((N_TYPES, R, D), lambda i: (0, i, 0)),
            pl.BlockSpec((N_TYPES, R, 1), lambda i: (0, i, 0)),
            pl.BlockSpec((R, D), lambda i: (i, 0)),
            pl.BlockSpec((N_TYPES, D, D), lambda i: (0, 0, 0)),
            pl.BlockSpec((N_TYPES, D, D), lambda i: (0, 0, 0)),
            pl.BlockSpec((N_TYPES, D), lambda i: (0, 0)),
        ],
        out_specs=pl.BlockSpec((R, D), lambda i: (i, 0)),
        out_shape=jax.ShapeDtypeStruct((N_NODES, D), jnp.float32),
    )(acc, deg3, x, w_neigh, w_self, b)


def kernel(x, edge_index_c2c, edge_index_c2d, edge_index_c2e, edge_index_codoc,
           edge_index_comention, edge_index_d2e, edge_index_ent,
           W_neigh, W_self, b):
    edge_lists = [edge_index_c2c, edge_index_c2d, edge_index_c2e,
                  edge_index_codoc, edge_index_comention, edge_index_d2e,
                  edge_index_ent]
    src = jnp.stack([ei[0] for ei in edge_lists])
    dst = jnp.stack([ei[1] for ei in edge_lists])
    pad = E_PAD - E_PER_TYPE
    src_p = jnp.pad(src, ((0, 0), (0, pad)))
    dst_p = jnp.pad(dst, ((0, 0), (0, pad)), constant_values=TRASH)
    src_r = src_p.reshape(N_TYPES, NS, NBLK, BLK)
    dst_r = dst_p.reshape(N_TYPES, NS, NBLK, BLK)

    zer128 = jnp.zeros((BLK, D), jnp.float32)
    zer640 = jnp.zeros((RPT,), jnp.float32)
    one128 = jnp.ones((BLK,), jnp.float32)

    acc, deg = _sc_segment_sums(x, src_r, dst_r, zer128, zer640, one128)
    deg3 = deg.reshape(N_TYPES, NROWS, 1)
    return _tc_combine(acc, deg3, x, W_neigh, W_self, b)
